# baseline (speedup 1.0000x reference)
import functools

import jax
import jax.numpy as jnp
from jax import lax
from jax.experimental import pallas as pl
from jax.experimental.pallas import tpu as pltpu, tpu_sc as plsc

N_V = 10000
N_V_PAD = 10240
E = 320000
D = 16
CHUNK = 128
N_CHUNKS = E // CHUNK
NC, NS = 2, 16
EH_ROWS = N_CHUNKS // 2
BLK = 125
NBLK = EH_ROWS // BLK
GROUPS = BLK * CHUNK // D

_mesh = plsc.VectorSubcoreMesh(core_axis_name="c", subcore_axis_name="s")


@functools.partial(
    pl.kernel,
    out_type=jax.ShapeDtypeStruct((D, N_V), jnp.float32),
    mesh=_mesh,
    compiler_params=pltpu.CompilerParams(
        use_tc_tiling_on_sc=False, needs_layout_passes=False),
    scratch_types=[
        pltpu.VMEM((2, BLK, CHUNK), jnp.int32),
        pltpu.VMEM((2, BLK, CHUNK), jnp.float32),
        pltpu.VMEM((N_V_PAD,), jnp.float32),
        pltpu.VMEM((N_V_PAD,), jnp.float32),
        pltpu.VMEM_SHARED((NS, N_V_PAD), jnp.float32),
        pltpu.VMEM_SHARED((N_CHUNKS, CHUNK), jnp.int32),
        pltpu.SemaphoreType.DMA,
        pltpu.SemaphoreType.DMA,
        pltpu.SemaphoreType.DMA,
        pltpu.SemaphoreType.DMA,
    ],
)
def _scatter_sc(idx_hbm, edge_hbm, out_hbm, idx_v, val_v, acc_v, peer_v,
                xch, idx_spm, sem_f0, sem_f1, sem_i0, sem_i1):
    c = lax.axis_index("c")
    s = lax.axis_index("s")
    f = s % 8
    h = s // 8
    row0 = h * EH_ROWS
    sem_f = (sem_f0, sem_f1)
    sem_i = (sem_i0, sem_i1)

    SROWS = N_CHUNKS // NS
    SLEFT = N_CHUNKS - SROWS * NS
    sr0 = pl.multiple_of(s * SROWS, 4)
    pltpu.sync_copy(idx_hbm.at[pl.ds(sr0, SROWS), 1, :],
                    idx_spm.at[pl.ds(sr0, SROWS)])

    @pl.when(s < SLEFT)
    def _():
        pltpu.sync_copy(idx_hbm.at[NS * SROWS + s, 1, :],
                        idx_spm.at[NS * SROWS + s])

    zero = jnp.zeros((D,), jnp.float32)

    @plsc.parallel_loop(0, N_V_PAD // D, unroll=8)
    def _z(i):
        acc_v[pl.ds(i * D, D)] = zero

    plsc.subcore_barrier()

    def _fill(blk, b):
        r = row0 + blk * BLK
        pltpu.async_copy(idx_spm.at[pl.ds(r, BLK)], idx_v.at[b],
                         sem_i[b])
        pltpu.async_copy(edge_hbm.at[c, pl.ds(r, BLK), f, :], val_v.at[b],
                         sem_f[b])

    def _wait_fill(b):
        pltpu.make_async_copy(idx_spm.at[pl.ds(0, BLK)],
                              idx_v.at[b], sem_i[b]).wait()
        pltpu.make_async_copy(edge_hbm.at[0, pl.ds(0, BLK), 0, :],
                              val_v.at[b], sem_f[b]).wait()

    _fill(0, 0)

    def _pair(p, carry):
        for b in range(2):
            k = p * 2 + b
            _wait_fill(b)

            @pl.when(k + 1 < NBLK)
            def _():
                _fill(k + 1, 1 - b)

            @plsc.parallel_loop(0, BLK, unroll=5)
            def _row(r):
                for j in range(CHUNK // D):
                    sl = pl.ds(j * D, D)
                    plsc.addupdate_scatter(
                        acc_v, [idx_v[b, r, sl]], val_v[b, r, sl])

        return carry

    lax.fori_loop(0, NBLK // 2, _pair, 0)

    pltpu.sync_copy(acc_v, xch.at[s])
    plsc.subcore_barrier()

    @pl.when(h == 0)
    def _():
        pltpu.sync_copy(xch.at[s + 8], peer_v)

        @plsc.parallel_loop(0, N_V_PAD // D, unroll=8)
        def _add(i):
            sl = pl.ds(i * D, D)
            acc_v[sl] = acc_v[sl] + peer_v[sl]

        pltpu.sync_copy(acc_v.at[pl.ds(0, N_V)],
                        out_hbm.at[c * 8 + f])


def kernel(vertex_attr, edgeij_pair, edge_attr, g, batch):
    idx = (edgeij_pair.astype(jnp.int32)
           .reshape(2, N_CHUNKS, CHUNK).transpose(1, 0, 2))
    edges = (edge_attr.T.reshape(2, 8, N_CHUNKS, CHUNK)
             .transpose(0, 2, 1, 3))
    out_t = _scatter_sc(idx, edges)
    return out_t.T
